# baseline (speedup 1.0000x reference)
import functools

import jax
import jax.numpy as jnp
from jax import lax
from jax.experimental import pallas as pl
from jax.experimental.pallas import tpu as pltpu
from jax.experimental.pallas import tpu_sc as plsc

BATCH = 16384
FEAT = 32
NC, NS, L = 2, 16, 16
NW = NC * NS
BPW = BATCH // NW
GCH = 128
NG = BPW // GCH


def _sc_partials(features, labels, centers):
    mesh = plsc.VectorSubcoreMesh(core_axis_name="c", subcore_axis_name="s")

    @functools.partial(
        pl.kernel,
        mesh=mesh,
        out_type=jax.ShapeDtypeStruct((NW, L), jnp.float32),
        scratch_types=[
            pltpu.VMEM((NG, GCH), jnp.int32),
            pltpu.VMEM((NG, GCH, FEAT), jnp.float32),
            pltpu.VMEM((NG, GCH, FEAT), jnp.float32),
            pltpu.VMEM((L,), jnp.float32),
            pltpu.SemaphoreType.DMA,
        ],
        compiler_params=pltpu.CompilerParams(use_tc_tiling_on_sc=False),
    )
    def k(feat_hbm, lab_hbm, cent_hbm, out_hbm, idx_v, feat_v, rows_v, acc_v, sem):
        wid = lax.axis_index("s") * NC + lax.axis_index("c")
        base = wid * BPW
        for g in range(NG):
            pltpu.sync_copy(lab_hbm.at[pl.ds(base + g * GCH, GCH)], idx_v.at[g])
        cps = [pltpu.make_async_copy(
                   feat_hbm.at[pl.ds(base + g * GCH, GCH), :], feat_v.at[g], sem)
               for g in range(NG)]
        cps += [pltpu.make_async_copy(cent_hbm.at[idx_v.at[g]], rows_v.at[g], sem)
                for g in range(NG)]
        for cp in cps:
            cp.start()
        for cp in cps:
            cp.wait()

        def body(i, acc):
            for g in range(NG):
                f0 = feat_v[g, i, pl.ds(0, L)]
                c0 = rows_v[g, i, pl.ds(0, L)]
                f1 = feat_v[g, i, pl.ds(L, L)]
                c1 = rows_v[g, i, pl.ds(L, L)]
                d0 = f0 - c0
                d1 = f1 - c1
                acc = acc + d0 * d0 + d1 * d1
            return acc

        acc = lax.fori_loop(0, GCH, body, jnp.zeros((L,), jnp.float32))
        acc_v[...] = acc
        pltpu.sync_copy(acc_v, out_hbm.at[wid])

    return k(features, labels, centers)


def _tc_reduce(partials):
    def body(p_ref, o_ref):
        o_ref[0, 0] = jnp.sum(p_ref[...]) * (1.0 / BATCH)

    out = pl.pallas_call(
        body,
        out_shape=jax.ShapeDtypeStruct((1, 1), jnp.float32),
        out_specs=pl.BlockSpec(memory_space=pltpu.SMEM),
    )(partials)
    return out.reshape(())


def kernel(features, labels, centers):
    labels = labels.astype(jnp.int32)
    partials = _sc_partials(features, labels, centers)
    return _tc_reduce(partials)


# speedup vs baseline: 3.5044x; 3.5044x over previous
import functools

import jax
import jax.numpy as jnp
from jax import lax
from jax.experimental import pallas as pl
from jax.experimental.pallas import tpu as pltpu
from jax.experimental.pallas import tpu_sc as plsc

BATCH = 16384
FEAT = 32
NUM_CLASSES = 1000000
NC, NS, L = 2, 16, 16
NW = NC * NS
BPW = BATCH // NW
RING = 8
DEPTH = 4
LAST_BASE = NUM_CLASSES - 128


def _sc_partials(features_t, labels, centers_t):
    mesh = plsc.VectorSubcoreMesh(core_axis_name="c", subcore_axis_name="s")

    @functools.partial(
        pl.kernel,
        mesh=mesh,
        out_type=jax.ShapeDtypeStruct((NW * L,), jnp.float32),
        scratch_types=[
            pltpu.VMEM((BPW,), jnp.int32),
            pltpu.VMEM((FEAT, BPW), jnp.float32),
            pltpu.VMEM((RING, FEAT, 128), jnp.float32),
            pltpu.VMEM((L,), jnp.float32),
            pltpu.SemaphoreType.DMA,
            pltpu.SemaphoreType.DMA,
        ],
        compiler_params=pltpu.CompilerParams(
            use_tc_tiling_on_sc=True, needs_layout_passes=False),
    )
    def k(feat_hbm, lab_hbm, cent_hbm, out_hbm, idx_v, feat_v, ring_v, acc_v,
          fsem, gsem):
        wid = lax.axis_index("s") * NC + lax.axis_index("c")
        base = pl.multiple_of(wid * BPW, 128)
        pltpu.sync_copy(lab_hbm.at[pl.ds(base, BPW)], idx_v)
        fcp = pltpu.make_async_copy(
            feat_hbm.at[:, pl.ds(base, BPW)], feat_v, fsem)
        fcp.start()

        fidx = jax.lax.broadcasted_iota(jnp.int32, (L,), 0)

        def col_base(r):
            return jnp.minimum((r >> 7) << 7, LAST_BASE)

        def fire(r, slot):
            pltpu.make_async_copy(
                cent_hbm.at[:, pl.ds(pl.multiple_of(col_base(r), 128), 128)],
                ring_v.at[slot],
                gsem,
            ).start()

        def drain_one(slot):
            pltpu.make_async_copy(
                cent_hbm.at[:, pl.ds(0, 128)],
                ring_v.at[slot],
                gsem,
            ).wait()

        lab0 = idx_v[pl.ds(0, L)]
        for j in range(DEPTH):
            fire(lab0[j], j)
        fcp.wait()

        def body(g, acc):
            j0 = g * L
            lab_vec = idx_v[pl.ds(j0, L)]
            nxt_off = jnp.minimum(j0 + L, BPW - L)
            lab_nxt = idx_v[pl.ds(nxt_off, L)]
            for jj in range(L):
                j = j0 + jj
                slot = j % RING
                drain_one(slot)
                r = lab_vec[jj]
                lane = jnp.broadcast_to(r - col_base(r), (L,))
                c0 = plsc.load_gather(ring_v.at[slot], [fidx, lane])
                c1 = plsc.load_gather(ring_v.at[slot], [fidx + L, lane])
                col = jnp.broadcast_to(j, (L,))
                f0 = plsc.load_gather(feat_v, [fidx, col])
                f1 = plsc.load_gather(feat_v, [fidx + L, col])
                d0 = f0 - c0
                d1 = f1 - c1
                acc = acc + d0 * d0 + d1 * d1
                if jj + DEPTH < L:
                    r_nxt = lab_vec[jj + DEPTH]
                else:
                    r_nxt = lab_nxt[jj + DEPTH - L]
                nxt = j + DEPTH

                @pl.when(nxt < BPW)
                def _():
                    fire(r_nxt, nxt % RING)

            return acc

        acc = lax.fori_loop(0, BPW // L, body, jnp.zeros((L,), jnp.float32))
        acc_v[...] = acc
        pltpu.sync_copy(acc_v, out_hbm.at[pl.ds(wid * L, L)])

    return k(features_t, labels, centers_t)


def _tc_reduce(partials):
    def body(p_ref, o_ref):
        o_ref[0, 0] = jnp.sum(p_ref[...]) * (1.0 / BATCH)

    out = pl.pallas_call(
        body,
        out_shape=jax.ShapeDtypeStruct((1, 1), jnp.float32),
        out_specs=pl.BlockSpec(memory_space=pltpu.SMEM),
    )(partials)
    return out.reshape(())


def kernel(features, labels, centers):
    labels = labels.astype(jnp.int32)
    partials = _sc_partials(features.T, labels, centers.T)
    return _tc_reduce(partials)


# speedup vs baseline: 4.6276x; 1.3205x over previous
import functools

import jax
import jax.numpy as jnp
from jax import lax
from jax.experimental import pallas as pl
from jax.experimental.pallas import tpu as pltpu
from jax.experimental.pallas import tpu_sc as plsc

BATCH = 16384
FEAT = 32
NUM_CLASSES = 1000000
NC, NS, L = 2, 16, 16
NW = NC * NS
BPW = BATCH // NW
RING = 16
DEPTH = 12
LAST_BASE = NUM_CLASSES - 128


def _sc_partials(features_t, labels, centers_t):
    mesh = plsc.VectorSubcoreMesh(core_axis_name="c", subcore_axis_name="s")

    @functools.partial(
        pl.kernel,
        mesh=mesh,
        out_type=jax.ShapeDtypeStruct((NW * L,), jnp.float32),
        scratch_types=[
            pltpu.VMEM((BPW,), jnp.int32),
            pltpu.VMEM((FEAT, BPW), jnp.float32),
            pltpu.VMEM((RING, FEAT, 128), jnp.float32),
            pltpu.VMEM((L,), jnp.float32),
            pltpu.SemaphoreType.DMA,
            pltpu.SemaphoreType.DMA,
        ],
        compiler_params=pltpu.CompilerParams(
            use_tc_tiling_on_sc=True, needs_layout_passes=False),
    )
    def k(feat_hbm, lab_hbm, cent_hbm, out_hbm, idx_v, feat_v, ring_v, acc_v,
          fsem, gsem):
        wid = lax.axis_index("s") * NC + lax.axis_index("c")
        base = pl.multiple_of(wid * BPW, 128)
        pltpu.sync_copy(lab_hbm.at[pl.ds(base, BPW)], idx_v)
        fcp = pltpu.make_async_copy(
            feat_hbm.at[:, pl.ds(base, BPW)], feat_v, fsem)
        fcp.start()

        fidx = jax.lax.broadcasted_iota(jnp.int32, (L,), 0)

        def col_base(r):
            return jnp.minimum((r >> 7) << 7, LAST_BASE)

        def fire(r, slot):
            pltpu.make_async_copy(
                cent_hbm.at[:, pl.ds(pl.multiple_of(col_base(r), 128), 128)],
                ring_v.at[slot],
                gsem,
            ).start()

        def drain_one(slot):
            pltpu.make_async_copy(
                cent_hbm.at[:, pl.ds(0, 128)],
                ring_v.at[slot],
                gsem,
            ).wait()

        lab0 = idx_v[pl.ds(0, L)]
        for j in range(DEPTH):
            fire(lab0[j], j)
        fcp.wait()

        def body(g, acc):
            j0 = g * L
            lab_vec = idx_v[pl.ds(j0, L)]
            nxt_off = jnp.minimum(j0 + L, BPW - L)
            lab_nxt = idx_v[pl.ds(nxt_off, L)]
            for jj in range(L):
                j = j0 + jj
                slot = j % RING
                drain_one(slot)
                r = lab_vec[jj]
                lane = jnp.broadcast_to(r - col_base(r), (L,))
                c0 = plsc.load_gather(ring_v.at[slot], [fidx, lane])
                c1 = plsc.load_gather(ring_v.at[slot], [fidx + L, lane])
                col = jnp.broadcast_to(j, (L,))
                f0 = plsc.load_gather(feat_v, [fidx, col])
                f1 = plsc.load_gather(feat_v, [fidx + L, col])
                d0 = f0 - c0
                d1 = f1 - c1
                acc = acc + d0 * d0 + d1 * d1
                if jj + DEPTH < L:
                    r_nxt = lab_vec[jj + DEPTH]
                else:
                    r_nxt = lab_nxt[jj + DEPTH - L]
                nxt = j + DEPTH

                @pl.when(nxt < BPW)
                def _():
                    fire(r_nxt, nxt % RING)

            return acc

        acc = lax.fori_loop(0, BPW // L, body, jnp.zeros((L,), jnp.float32))
        acc_v[...] = acc
        pltpu.sync_copy(acc_v, out_hbm.at[pl.ds(wid * L, L)])

    return k(features_t, labels, centers_t)


def _tc_reduce(partials):
    def body(p_ref, o_ref):
        o_ref[0, 0] = jnp.sum(p_ref[...]) * (1.0 / BATCH)

    out = pl.pallas_call(
        body,
        out_shape=jax.ShapeDtypeStruct((1, 1), jnp.float32),
        out_specs=pl.BlockSpec(memory_space=pltpu.SMEM),
    )(partials)
    return out.reshape(())


def kernel(features, labels, centers):
    labels = labels.astype(jnp.int32)
    partials = _sc_partials(features.T, labels, centers.T)
    return _tc_reduce(partials)
